# baseline (speedup 1.0000x reference)
import functools

import jax
import jax.numpy as jnp
from jax import lax
from jax.experimental import pallas as pl
from jax.experimental.pallas import tpu as pltpu
from jax.experimental.pallas import tpu_sc as plsc

N_NODES = 10000
N_EDGES = 320000
D = 128
NC = 2
NS = 16
NW = NC * NS
C = 128
CH = N_EDGES // C
GMAX = CH // NW + 1
N_PAD = 10240
RPT = N_PAD // NS
CNT_PAD = 10240


def _sc_segment_sum(x, e4, z_acc, z_cnt):
    mesh = plsc.VectorSubcoreMesh(
        core_axis_name="c", subcore_axis_name="s", num_cores=NC, num_subcores=NS
    )

    @functools.partial(
        pl.kernel,
        out_type=[
            jax.ShapeDtypeStruct((NC, N_PAD, D), jnp.float32),
            jax.ShapeDtypeStruct((NC, CNT_PAD), jnp.float32),
        ],
        mesh=mesh,
        scratch_types=[
            pltpu.VMEM((GMAX, C), jnp.int32),
            pltpu.VMEM((2, C), jnp.int32),
            pltpu.VMEM((C, D), jnp.float32),
            pltpu.VMEM((C, D), jnp.float32),
            pltpu.VMEM((C,), jnp.float32),
            pltpu.VMEM_SHARED((N_PAD, D), jnp.float32),
            pltpu.VMEM_SHARED((CNT_PAD,), jnp.float32),
            pltpu.SemaphoreType.DMA,
            pltpu.SemaphoreType.DMA,
            pltpu.SemaphoreType.DMA,
            pltpu.SemaphoreType.DMA,
            pltpu.SemaphoreType.DMA,
            pltpu.SemaphoreType.DMA,
            pltpu.SemaphoreType.DMA,
            pltpu.SemaphoreType.DMA,
        ],
    )
    def sc_kernel(x_hbm, e_hbm, zacc_hbm, zcnt_hbm,
                  acc_out, cnt_out, dst_v, sidx_v, rows0_v, rows1_v, ones_v,
                  acc_sh, cnt_sh, semi0, semi1, semg0, semg1, semc, semd,
                  sems0, sems1):
        c = lax.axis_index("c")
        s = lax.axis_index("s")
        wid = c * NS + s
        nch = jnp.where(wid < CH - (GMAX - 1) * NW, GMAX, GMAX - 1)

        def dload(g, carry):
            pltpu.async_copy(e_hbm.at[1, pl.ds((g * NW + wid) * C, C)],
                             dst_v.at[g], semd)
            return carry

        def ddrain(g, carry):
            pltpu.make_async_copy(e_hbm.at[1, pl.ds(0, C)], dst_v.at[0],
                                  semd).wait()
            return carry

        lax.fori_loop(0, nch, dload, 0)

        pltpu.sync_copy(zacc_hbm, acc_sh.at[pl.ds(s * RPT, RPT)])

        @pl.when(s == 0)
        def _():
            pltpu.sync_copy(zcnt_hbm, cnt_sh)

        for j in range(C // 16):
            ones_v[pl.ds(j * 16, 16)] = jnp.full((16,), 1.0, jnp.float32)
        lax.fori_loop(0, nch, ddrain, 0)

        def istart(g, slot, sem):
            pltpu.async_copy(e_hbm.at[0, pl.ds((g * NW + wid) * C, C)],
                             slot, sem)

        def idrain(slot, sem):
            pltpu.make_async_copy(e_hbm.at[0, pl.ds(0, C)], slot, sem).wait()

        def gstart(slot, buf, sem):
            pltpu.async_copy(x_hbm.at[slot], buf, sem)

        def gdrain(buf, sem):
            pltpu.make_async_copy(x_hbm.at[pl.ds(0, C)], buf, sem).wait()

        def sstart(buf, g, sem):
            pltpu.async_copy(buf, acc_sh.at[dst_v.at[g]], sem, add=True)
            pltpu.async_copy(ones_v, cnt_sh.at[dst_v.at[g]], semc, add=True)

        def sdrain(buf, sem):
            pltpu.make_async_copy(x_hbm.at[pl.ds(0, C)], buf, sem).wait()

        istart(0, sidx_v.at[0], semi0)
        idrain(sidx_v.at[0], semi0)
        gstart(sidx_v.at[0], rows0_v, semg0)
        istart(1, sidx_v.at[1], semi1)

        plsc.subcore_barrier()

        def chunk2(k, carry):
            g = 2 * k
            gdrain(rows0_v, semg0)

            @pl.when(g + 2 < nch)
            def _():
                istart(g + 2, sidx_v.at[0], semi0)

            sstart(rows0_v, g, sems0)

            @pl.when(k > 0)
            def _():
                sdrain(rows1_v, sems1)

            idrain(sidx_v.at[1], semi1)
            gstart(sidx_v.at[1], rows1_v, semg1)
            gdrain(rows1_v, semg1)

            @pl.when(g + 3 < nch)
            def _():
                istart(g + 3, sidx_v.at[1], semi1)

            sstart(rows1_v, g + 1, sems1)
            sdrain(rows0_v, sems0)

            @pl.when(g + 2 < nch)
            def _():
                idrain(sidx_v.at[0], semi0)
                gstart(sidx_v.at[0], rows0_v, semg0)

            return carry

        lax.fori_loop(0, (GMAX - 1) // 2, chunk2, 0)

        @pl.when(nch == GMAX)
        def _():
            gdrain(rows0_v, semg0)
            sstart(rows0_v, GMAX - 1, sems0)
            sdrain(rows0_v, sems0)

        sdrain(rows1_v, sems1)

        def cdrain(g, carry):
            pltpu.make_async_copy(zcnt_hbm.at[pl.ds(0, C)], ones_v, semc).wait()
            return carry

        lax.fori_loop(0, nch, cdrain, 0)


        plsc.subcore_barrier()

        pltpu.sync_copy(acc_sh.at[pl.ds(s * RPT, RPT)],
                        acc_out.at[c, pl.ds(s * RPT, RPT)])

        @pl.when(s == 0)
        def _():
            pltpu.sync_copy(cnt_sh, cnt_out.at[c])

    return sc_kernel(x, e4, z_acc, z_cnt)


def _tc_combine(part, cntb, x, wt, b2):
    BN = 1000
    grid = (N_NODES // BN,)

    def body(part_ref, cnt_ref, x_ref, wt_ref, b_ref, out_ref):
        neigh = (part_ref[0] + part_ref[1]) / cnt_ref[...]
        comb = jnp.concatenate([neigh, x_ref[...]], axis=1)
        o = lax.dot_general(comb, wt_ref[...], (((1,), (0,)), ((), ())),
                            preferred_element_type=jnp.float32)
        o = o + b_ref[...]
        out_ref[...] = o * jax.nn.sigmoid(o)

    return pl.pallas_call(
        body,
        grid=grid,
        in_specs=[
            pl.BlockSpec((NC, BN, D), lambda i: (0, i, 0)),
            pl.BlockSpec((BN, D), lambda i: (i, 0)),
            pl.BlockSpec((BN, D), lambda i: (i, 0)),
            pl.BlockSpec((2 * D, D), lambda i: (0, 0)),
            pl.BlockSpec((1, D), lambda i: (0, 0)),
        ],
        out_specs=pl.BlockSpec((BN, D), lambda i: (i, 0)),
        out_shape=jax.ShapeDtypeStruct((N_NODES, D), jnp.float32),
    )(part, cntb, x, wt, b2)


def kernel(x, edge_index, W, b):
    e4 = edge_index.astype(jnp.int32)
    z_acc = jnp.zeros((RPT, D), jnp.float32)
    z_cnt = jnp.zeros((CNT_PAD,), jnp.float32)

    part, cnt = _sc_segment_sum(x, e4, z_acc, z_cnt)

    cntb = jnp.broadcast_to(
        jnp.maximum(cnt[0, :N_NODES] + cnt[1, :N_NODES], 1.0)[:, None],
        (N_NODES, D))
    wt = W.T
    b2 = b.reshape(1, D)
    return _tc_combine(part, cntb, x, wt, b2)


# speedup vs baseline: 1.0149x; 1.0149x over previous
import functools

import jax
import jax.numpy as jnp
from jax import lax
from jax.experimental import pallas as pl
from jax.experimental.pallas import tpu as pltpu
from jax.experimental.pallas import tpu_sc as plsc

N_NODES = 10000
N_EDGES = 320000
D = 128
NC = 2
NS = 16
NW = NC * NS
C = 128
CH = N_EDGES // C
GMAX = CH // NW + 1
N_PAD = 10240
RPT = N_PAD // NS
CNT_PAD = 10240


def _sc_segment_sum(x, e4, z_acc, z_cnt):
    mesh = plsc.VectorSubcoreMesh(
        core_axis_name="c", subcore_axis_name="s", num_cores=NC, num_subcores=NS
    )

    @functools.partial(
        pl.kernel,
        out_type=[
            jax.ShapeDtypeStruct((NC, N_PAD, D), jnp.float32),
            jax.ShapeDtypeStruct((NC, CNT_PAD), jnp.float32),
        ],
        mesh=mesh,
        scratch_types=[
            pltpu.VMEM((GMAX, C), jnp.int32),
            pltpu.VMEM((2, C), jnp.int32),
            pltpu.VMEM((C, D), jnp.float32),
            pltpu.VMEM((C, D), jnp.float32),
            pltpu.VMEM((C,), jnp.float32),
            pltpu.VMEM_SHARED((N_PAD, D), jnp.float32),
            pltpu.VMEM_SHARED((CNT_PAD,), jnp.float32),
            pltpu.SemaphoreType.DMA,
            pltpu.SemaphoreType.DMA,
            pltpu.SemaphoreType.DMA,
            pltpu.SemaphoreType.DMA,
            pltpu.SemaphoreType.DMA,
            pltpu.SemaphoreType.DMA,
            pltpu.SemaphoreType.DMA,
            pltpu.SemaphoreType.DMA,
        ],
    )
    def sc_kernel(x_hbm, e_hbm, zacc_hbm, zcnt_hbm,
                  acc_out, cnt_out, dst_v, sidx_v, rows0_v, rows1_v, ones_v,
                  acc_sh, cnt_sh, semi0, semi1, semg0, semg1, semc, semd,
                  sems0, sems1):
        c = lax.axis_index("c")
        s = lax.axis_index("s")
        wid = c * NS + s
        nch = jnp.where(wid < CH - (GMAX - 1) * NW, GMAX, GMAX - 1)

        def dload(g, carry):
            pltpu.async_copy(e_hbm.at[1, pl.ds((g * NW + wid) * C, C)],
                             dst_v.at[g], semd)
            return carry

        def ddrain(g, carry):
            pltpu.make_async_copy(e_hbm.at[1, pl.ds(0, C)], dst_v.at[0],
                                  semd).wait()
            return carry

        lax.fori_loop(0, nch, dload, 0)

        pltpu.sync_copy(zacc_hbm, acc_sh.at[pl.ds(s * RPT, RPT)])

        @pl.when(s == 0)
        def _():
            pltpu.sync_copy(zcnt_hbm, cnt_sh)

        for j in range(C // 16):
            ones_v[pl.ds(j * 16, 16)] = jnp.full((16,), 1.0, jnp.float32)
        lax.fori_loop(0, nch, ddrain, 0)

        def istart(g, slot, sem):
            pltpu.async_copy(e_hbm.at[0, pl.ds((g * NW + wid) * C, C)],
                             slot, sem)

        def idrain(slot, sem):
            pltpu.make_async_copy(e_hbm.at[0, pl.ds(0, C)], slot, sem).wait()

        def gstart(slot, buf, sem):
            pltpu.async_copy(x_hbm.at[slot], buf, sem)

        def gdrain(buf, sem):
            pltpu.make_async_copy(x_hbm.at[pl.ds(0, C)], buf, sem).wait()

        def sstart(buf, g, sem):
            pltpu.async_copy(buf, acc_sh.at[dst_v.at[g]], sem, add=True)
            pltpu.async_copy(ones_v, cnt_sh.at[dst_v.at[g]], semc, add=True)

        def sdrain(buf, sem):
            pltpu.make_async_copy(x_hbm.at[pl.ds(0, C)], buf, sem).wait()

        istart(0, sidx_v.at[0], semi0)
        idrain(sidx_v.at[0], semi0)
        gstart(sidx_v.at[0], rows0_v, semg0)
        istart(1, sidx_v.at[1], semi1)

        plsc.subcore_barrier()

        def chunk2(k, carry):
            g = 2 * k
            gdrain(rows0_v, semg0)

            @pl.when(g + 2 < nch)
            def _():
                istart(g + 2, sidx_v.at[0], semi0)

            sstart(rows0_v, g, sems0)

            @pl.when(k > 0)
            def _():
                sdrain(rows1_v, sems1)

            idrain(sidx_v.at[1], semi1)
            gstart(sidx_v.at[1], rows1_v, semg1)
            gdrain(rows1_v, semg1)

            @pl.when(g + 3 < nch)
            def _():
                istart(g + 3, sidx_v.at[1], semi1)

            sstart(rows1_v, g + 1, sems1)
            sdrain(rows0_v, sems0)

            @pl.when(g + 2 < nch)
            def _():
                idrain(sidx_v.at[0], semi0)
                gstart(sidx_v.at[0], rows0_v, semg0)

            return carry

        lax.fori_loop(0, (GMAX - 1) // 2, chunk2, 0)

        @pl.when(nch == GMAX)
        def _():
            gdrain(rows0_v, semg0)
            sstart(rows0_v, GMAX - 1, sems0)
            sdrain(rows0_v, sems0)

        sdrain(rows1_v, sems1)

        def cdrain(g, carry):
            pltpu.make_async_copy(zcnt_hbm.at[pl.ds(0, C)], ones_v, semc).wait()
            return carry

        lax.fori_loop(0, nch, cdrain, 0)


        plsc.subcore_barrier()

        pltpu.sync_copy(acc_sh.at[pl.ds(s * RPT, RPT)],
                        acc_out.at[c, pl.ds(s * RPT, RPT)])

        @pl.when(s == 0)
        def _():
            pltpu.sync_copy(cnt_sh, cnt_out.at[c])

    return sc_kernel(x, e4, z_acc, z_cnt)


def _tc_combine(part, cntb, x, wt, b2):
    BN = 2000
    grid = (N_NODES // BN,)

    def body(part_ref, cnt_ref, x_ref, wt_ref, b_ref, out_ref):
        neigh = (part_ref[0] + part_ref[1]) / cnt_ref[...]
        comb = jnp.concatenate([neigh, x_ref[...]], axis=1)
        o = lax.dot_general(comb, wt_ref[...], (((1,), (0,)), ((), ())),
                            preferred_element_type=jnp.float32)
        o = o + b_ref[...]
        out_ref[...] = o * jax.nn.sigmoid(o)

    return pl.pallas_call(
        body,
        grid=grid,
        in_specs=[
            pl.BlockSpec((NC, BN, D), lambda i: (0, i, 0)),
            pl.BlockSpec((BN, D), lambda i: (i, 0)),
            pl.BlockSpec((BN, D), lambda i: (i, 0)),
            pl.BlockSpec((2 * D, D), lambda i: (0, 0)),
            pl.BlockSpec((1, D), lambda i: (0, 0)),
        ],
        out_specs=pl.BlockSpec((BN, D), lambda i: (i, 0)),
        out_shape=jax.ShapeDtypeStruct((N_NODES, D), jnp.float32),
    )(part, cntb, x, wt, b2)


def kernel(x, edge_index, W, b):
    e4 = edge_index.astype(jnp.int32)
    z_acc = jnp.zeros((RPT, D), jnp.float32)
    z_cnt = jnp.zeros((CNT_PAD,), jnp.float32)

    part, cnt = _sc_segment_sum(x, e4, z_acc, z_cnt)

    cntb = jnp.broadcast_to(
        jnp.maximum(cnt[0, :N_NODES] + cnt[1, :N_NODES], 1.0)[:, None],
        (N_NODES, D))
    wt = W.T
    b2 = b.reshape(1, D)
    return _tc_combine(part, cntb, x, wt, b2)
